# baseline (speedup 1.0000x reference)
import dataclasses
import functools

import jax
import jax.numpy as jnp
from jax import lax
from jax.experimental import pallas as pl
from jax.experimental.pallas import tpu as pltpu
from jax.experimental.pallas import tpu_sc as plsc

N_NODES = 10000
N_EDGES = 320000
HID = 128
NGRAPH = 16
BN_EPS_C = 1e-5

_NODE_BLOCK = 1000
_EDGE_BLOCK = 3200
_CHUNK = 64
_NCHUNK = N_EDGES // _CHUNK
_NTILE = 16
_NWORKER = 2 * _NTILE
_ROWS_MAIN = 624
_TAIL0 = _ROWS_MAIN * _NTILE
_TAIL = N_NODES - _TAIL0


def _edge_pack_body(x_ref, wa_ref, ba_ref, wb_ref, bb_ref, o_ref):
    acc_a = (
        jnp.dot(x_ref[...], wa_ref[...], preferred_element_type=jnp.float32)
        + ba_ref[...]
    )
    acc_b = (
        jnp.dot(x_ref[...], wb_ref[...], preferred_element_type=jnp.float32)
        + bb_ref[...]
    )
    lo = lax.bitcast_convert_type(acc_a.astype(jnp.bfloat16),
                                  jnp.uint16).astype(jnp.uint32)
    hi = lax.bitcast_convert_type(acc_b.astype(jnp.bfloat16),
                                  jnp.uint16).astype(jnp.uint32)
    o_ref[...] = (lo | (hi << 16)).astype(jnp.int32)


def _tc_edge_pack(x, WA, bA, WB, bB):
    n, k = x.shape
    m = WA.shape[1]
    return pl.pallas_call(
        _edge_pack_body,
        grid=(n // _EDGE_BLOCK,),
        in_specs=[
            pl.BlockSpec((_EDGE_BLOCK, k), lambda i: (i, 0)),
            pl.BlockSpec((k, m), lambda i: (0, 0)),
            pl.BlockSpec((1, m), lambda i: (0, 0)),
            pl.BlockSpec((k, m), lambda i: (0, 0)),
            pl.BlockSpec((1, m), lambda i: (0, 0)),
        ],
        out_specs=pl.BlockSpec((_EDGE_BLOCK, m), lambda i: (i, 0)),
        out_shape=jax.ShapeDtypeStruct((n, m), jnp.int32),
    )(x, WA, bA.reshape(1, -1), WB, bB.reshape(1, -1))


def _linear_body(x_ref, w_ref, b_ref, o_ref):
    acc = (
        jnp.dot(x_ref[...], w_ref[...], preferred_element_type=jnp.float32)
        + b_ref[...]
    )
    o_ref[...] = acc.astype(o_ref.dtype)


def _tc_linear(x, W, b, block_rows, out_dtype=jnp.float32):
    n, k = x.shape
    m = W.shape[1]
    return pl.pallas_call(
        _linear_body,
        grid=(n // block_rows,),
        in_specs=[
            pl.BlockSpec((block_rows, k), lambda i: (i, 0)),
            pl.BlockSpec((k, m), lambda i: (0, 0)),
            pl.BlockSpec((1, m), lambda i: (0, 0)),
        ],
        out_specs=pl.BlockSpec((block_rows, m), lambda i: (i, 0)),
        out_shape=jax.ShapeDtypeStruct((n, m), out_dtype),
    )(x, W, b)


def _node_mlp_body(h_ref, agg_ref, w1_ref, b1_ref, w2_ref, b2_ref,
                   g_ref, be_ref, mu_ref, var_ref, eps_ref, o_ref):
    a = agg_ref[0] + agg_ref[1]
    z = (1.0 + eps_ref[0]) * h_ref[...] + a
    u = jnp.maximum(
        jnp.dot(z, w1_ref[...], preferred_element_type=jnp.float32) + b1_ref[...],
        0.0,
    )
    v = jnp.dot(u, w2_ref[...], preferred_element_type=jnp.float32) + b2_ref[...]
    scale = g_ref[...] * lax.rsqrt(var_ref[...] + BN_EPS_C)
    shift = be_ref[...] - mu_ref[...] * scale
    o_ref[...] = jnp.maximum(v * scale + shift, 0.0)


def _tc_node_mlp(h, agg2, W1i, b1i, W2i, b2i, gi, bei, mui, vari, epsi):
    n = h.shape[0]
    nb = n // _NODE_BLOCK
    row = lambda a: a.reshape(1, -1)
    return pl.pallas_call(
        _node_mlp_body,
        grid=(nb,),
        in_specs=[
            pl.BlockSpec((_NODE_BLOCK, HID), lambda i: (i, 0)),
            pl.BlockSpec((2, _NODE_BLOCK, HID), lambda i: (0, i, 0)),
            pl.BlockSpec((HID, 2 * HID), lambda i: (0, 0)),
            pl.BlockSpec((1, 2 * HID), lambda i: (0, 0)),
            pl.BlockSpec((2 * HID, HID), lambda i: (0, 0)),
            pl.BlockSpec((1, HID), lambda i: (0, 0)),
            pl.BlockSpec((1, HID), lambda i: (0, 0)),
            pl.BlockSpec((1, HID), lambda i: (0, 0)),
            pl.BlockSpec((1, HID), lambda i: (0, 0)),
            pl.BlockSpec((1, HID), lambda i: (0, 0)),
            pl.BlockSpec(memory_space=pltpu.SMEM),
        ],
        out_specs=pl.BlockSpec((_NODE_BLOCK, HID), lambda i: (i, 0)),
        out_shape=jax.ShapeDtypeStruct((n, HID), jnp.float32),
    )(h, agg2, W1i, row(b1i), W2i, row(b2i), row(gi), row(bei), row(mui),
      row(vari), epsi.reshape(1))


def _pool_body(h_ref, b3_ref, wf_ref, bf_ref, o_ref, sums_ref, cnts_ref):
    i = pl.program_id(0)
    nb = pl.num_programs(0)

    @pl.when(i == 0)
    def _():
        sums_ref[...] = jnp.zeros_like(sums_ref)
        cnts_ref[...] = jnp.zeros_like(cnts_ref)

    seg = b3_ref[0]
    gids = lax.broadcasted_iota(jnp.int32, (NGRAPH, seg.shape[1]), 0)
    onehot = (gids == seg).astype(jnp.float32)
    sums_ref[...] += jnp.dot(onehot, h_ref[...],
                             preferred_element_type=jnp.float32)
    cnts_ref[...] += jnp.sum(onehot, axis=1, keepdims=True)

    @pl.when(i == nb - 1)
    def _():
        pooled = sums_ref[...] / jnp.clip(cnts_ref[...], 1.0, None)
        o_ref[...] = (
            jnp.dot(pooled, wf_ref[...], preferred_element_type=jnp.float32)
            + bf_ref[...]
        )


def _tc_pool(h, batch3, W_final, b_final):
    n = h.shape[0]
    nb = n // _NODE_BLOCK
    return pl.pallas_call(
        _pool_body,
        grid=(nb,),
        in_specs=[
            pl.BlockSpec((_NODE_BLOCK, HID), lambda i: (i, 0)),
            pl.BlockSpec((1, 1, _NODE_BLOCK), lambda i: (i, 0, 0)),
            pl.BlockSpec((HID, HID), lambda i: (0, 0)),
            pl.BlockSpec((1, HID), lambda i: (0, 0)),
        ],
        out_specs=pl.BlockSpec((NGRAPH, HID), lambda i: (0, 0)),
        out_shape=jax.ShapeDtypeStruct((NGRAPH, HID), jnp.float32),
        scratch_shapes=[
            pltpu.VMEM((NGRAPH, HID), jnp.float32),
            pltpu.VMEM((NGRAPH, 1), jnp.float32),
        ],
    )(h, batch3, W_final, b_final.reshape(1, -1))


_SC_MESH = plsc.VectorSubcoreMesh(core_axis_name="c", subcore_axis_name="s")

_SC_CP = pltpu.CompilerParams()
if "needs_layout_passes" in pltpu.CompilerParams.__dataclass_fields__:
    _SC_CP = dataclasses.replace(_SC_CP, needs_layout_passes=False)


_NPW = -(-_NCHUNK // _NWORKER)
_NPW3 = _NPW + (-_NPW) % 3


@functools.partial(
    pl.kernel,
    out_type=jax.ShapeDtypeStruct((2, N_NODES, HID), jnp.float32),
    mesh=_SC_MESH,
    compiler_params=_SC_CP,
    scratch_types=[
        pltpu.VMEM((_CHUNK,), jnp.int32),
        pltpu.VMEM((_CHUNK,), jnp.int32),
        pltpu.VMEM((_CHUNK,), jnp.int32),
        pltpu.VMEM((_CHUNK,), jnp.int32),
        pltpu.VMEM((_CHUNK,), jnp.int32),
        pltpu.VMEM((_CHUNK,), jnp.int32),
        pltpu.VMEM((_CHUNK,), jnp.int32),
        pltpu.VMEM((_CHUNK,), jnp.int32),
        pltpu.VMEM((_CHUNK,), jnp.int32),
        pltpu.VMEM((3, _CHUNK, HID), jnp.float32),
        pltpu.VMEM((3, _CHUNK, HID // 2), jnp.int32),
        pltpu.VMEM_SHARED((N_NODES, HID), jnp.float32),
        pltpu.SemaphoreType.DMA,
        pltpu.SemaphoreType.DMA,
        pltpu.SemaphoreType.DMA,
        pltpu.SemaphoreType.DMA,
        pltpu.SemaphoreType.DMA,
        pltpu.SemaphoreType.DMA,
        pltpu.SemaphoreType.DMA,
        pltpu.SemaphoreType.DMA,
        pltpu.SemaphoreType.DMA,
        pltpu.SemaphoreType.DMA,
        pltpu.SemaphoreType.DMA,
        pltpu.SemaphoreType.DMA,
    ],
)
def _sc_msg_agg(h_hbm, e_hbm, src_hbm, dst_hbm, zeros_hbm, out_hbm,
                sr0, sr1, sr2, dr0, dr1, dr2, dn0, dn1, dn2,
                rows_v, e_v, agg_sh,
                si0, si1, si2, sg0, sg1, sg2, se0, se1, se2, ss0, ss1, ss2):
    srcin = (sr0, sr1, sr2)
    dstin = (dr0, dr1, dr2)
    dsts = (dn0, dn1, dn2)
    cid = lax.axis_index("c")
    sid = lax.axis_index("s")
    wid = cid * _NTILE + sid
    row0 = sid * _ROWS_MAIN

    si = (si0, si1, si2)
    sg = (sg0, sg1, sg2)
    se = (se0, se1, se2)
    ss = (ss0, ss1, ss2)

    pltpu.sync_copy(zeros_hbm, agg_sh.at[pl.ds(row0, _ROWS_MAIN)])

    @pl.when(sid == _NTILE - 1)
    def _():
        pltpu.sync_copy(zeros_hbm.at[pl.ds(0, _TAIL)],
                        agg_sh.at[pl.ds(_TAIL0, _TAIL)])

    plsc.subcore_barrier()

    def chunk_of(q):
        return q * _NWORKER + wid

    def idx_copies(q, b):
        base = chunk_of(q) * _CHUNK
        return (
            pltpu.make_async_copy(src_hbm.at[pl.ds(base, _CHUNK)],
                                  srcin[b], si[b]),
            pltpu.make_async_copy(dst_hbm.at[pl.ds(base, _CHUNK)],
                                  dstin[b], si[b]),
        )

    def issue_idx(q, b):
        for cp in idx_copies(q, b):
            cp.start()

    def wait_idx(q, b):
        for cp in idx_copies(q, b):
            cp.wait()

    def issue_body(q, b):
        pltpu.async_copy(h_hbm.at[srcin[b]], rows_v.at[b], sg[b])
        pltpu.async_copy(
            e_hbm.at[pl.ds(chunk_of(q) * _CHUNK, _CHUNK)], e_v.at[b], se[b])

    def wait_scatter(b):
        pltpu.make_async_copy(
            rows_v.at[b], agg_sh.at[dsts[b]], ss[b]).wait()

    issue_idx(0, 0)
    wait_idx(0, 0)
    issue_body(0, 0)
    issue_idx(1, 1)
    issue_idx(2, 2)

    def substep(q, b):
        b1 = (b + 1) % 3

        @pl.when(chunk_of(q + 1) < _NCHUNK)
        def _():
            wait_idx(q + 1, b1)

            @pl.when(q >= 2)
            def _():
                wait_scatter(b1)

            issue_body(q + 1, b1)

        @pl.when(chunk_of(q) < _NCHUNK)
        def _():
            pltpu.make_async_copy(h_hbm.at[srcin[b]], rows_v.at[b],
                                  sg[b]).wait()
            pltpu.make_async_copy(
                e_hbm.at[pl.ds(chunk_of(q) * _CHUNK, _CHUNK)], e_v.at[b],
                se[b]).wait()

            for k in range(_CHUNK // 16):
                sl = pl.ds(k * 16, 16)
                dsts[b][sl] = dstin[b][sl]

            @pl.loop(0, _CHUNK)
            def _(r):
                for k in range(HID // 32):
                    ew = e_v[b, r, pl.ds(k * 16, 16)]
                    ebf = plsc.bitcast(ew, jnp.bfloat16)
                    e0, e1 = plsc.unpack(
                        ebf, format=plsc.PackFormat.INTERLEAVED)
                    lo = pl.ds(k * 32, 16)
                    hi = pl.ds(k * 32 + 16, 16)
                    rows_v[b, r, lo] = jnp.maximum(rows_v[b, r, lo] + e0, 0.0)
                    rows_v[b, r, hi] = jnp.maximum(rows_v[b, r, hi] + e1, 0.0)

            pltpu.async_copy(rows_v.at[b], agg_sh.at[dsts[b]],
                             ss[b], add=True)

        @pl.when(chunk_of(q + 3) < _NCHUNK)
        def _():
            issue_idx(q + 3, b)

    @pl.loop(0, _NPW3, step=3)
    def _(q):
        substep(q, 0)
        substep(q + 1, 1)
        substep(q + 2, 2)

    for b in range(3):
        wait_scatter(b)

    plsc.subcore_barrier()
    pltpu.sync_copy(agg_sh.at[pl.ds(row0, _ROWS_MAIN)],
                    out_hbm.at[cid, pl.ds(row0, _ROWS_MAIN)])

    @pl.when(sid == _NTILE - 1)
    def _():
        pltpu.sync_copy(agg_sh.at[pl.ds(_TAIL0, _TAIL)],
                        out_hbm.at[cid, pl.ds(_TAIL0, _TAIL)])


def kernel(x, edge_index, batch, edge_attr, W_init, b_init, eps,
           W_edge, b_edge, W1, b1, W2, b2, bn_gamma, bn_beta, bn_mean,
           bn_var, W_final, b_final):
    eidx = edge_index.astype(jnp.int32)
    src_ids = eidx[0]
    dst_ids = eidx[1]
    batch3 = batch.astype(jnp.int32).reshape(
        N_NODES // _NODE_BLOCK, 1, _NODE_BLOCK)
    zeros = jnp.zeros((_ROWS_MAIN, HID), jnp.float32)

    cols_a = []
    cols_b = []
    for g in range(HID // 32):
        cols_a.extend(range(g * 32, g * 32 + 16))
        cols_b.extend(range(g * 32 + 16, g * 32 + 32))
    cols_a = jnp.asarray(cols_a, dtype=jnp.int32)
    cols_b = jnp.asarray(cols_b, dtype=jnp.int32)

    h = _tc_linear(x, W_init, b_init.reshape(1, -1), _NODE_BLOCK)
    for i in range(W_edge.shape[0]):
        e = _tc_edge_pack(edge_attr, W_edge[i][:, cols_a], b_edge[i][cols_a],
                          W_edge[i][:, cols_b], b_edge[i][cols_b])
        agg2 = _sc_msg_agg(h, e, src_ids, dst_ids, zeros)
        h = _tc_node_mlp(h, agg2, W1[i], b1[i], W2[i], b2[i],
                         bn_gamma[i], bn_beta[i], bn_mean[i], bn_var[i],
                         eps[i])
    return _tc_pool(h, batch3, W_final, b_final)


# speedup vs baseline: 1.0011x; 1.0011x over previous
import dataclasses
import functools

import jax
import jax.numpy as jnp
from jax import lax
from jax.experimental import pallas as pl
from jax.experimental.pallas import tpu as pltpu
from jax.experimental.pallas import tpu_sc as plsc

N_NODES = 10000
N_EDGES = 320000
HID = 128
NGRAPH = 16
BN_EPS_C = 1e-5

_NODE_BLOCK = 1000
_EDGE_BLOCK = 3200
_CHUNK = 64
_NCHUNK = N_EDGES // _CHUNK
_NTILE = 16
_NWORKER = 2 * _NTILE
_ROWS_MAIN = 624
_TAIL0 = _ROWS_MAIN * _NTILE
_TAIL = N_NODES - _TAIL0


def _edge_pack_body(x_ref, wa_ref, ba_ref, wb_ref, bb_ref, o_ref):
    acc_a = (
        jnp.dot(x_ref[...], wa_ref[...], preferred_element_type=jnp.float32)
        + ba_ref[...]
    )
    acc_b = (
        jnp.dot(x_ref[...], wb_ref[...], preferred_element_type=jnp.float32)
        + bb_ref[...]
    )
    lo = lax.bitcast_convert_type(acc_a.astype(jnp.bfloat16),
                                  jnp.uint16).astype(jnp.uint32)
    hi = lax.bitcast_convert_type(acc_b.astype(jnp.bfloat16),
                                  jnp.uint16).astype(jnp.uint32)
    o_ref[...] = (lo | (hi << 16)).astype(jnp.int32)


def _tc_edge_pack(x, WA, bA, WB, bB):
    n, k = x.shape
    m = WA.shape[1]
    return pl.pallas_call(
        _edge_pack_body,
        grid=(n // _EDGE_BLOCK,),
        in_specs=[
            pl.BlockSpec((_EDGE_BLOCK, k), lambda i: (i, 0)),
            pl.BlockSpec((k, m), lambda i: (0, 0)),
            pl.BlockSpec((1, m), lambda i: (0, 0)),
            pl.BlockSpec((k, m), lambda i: (0, 0)),
            pl.BlockSpec((1, m), lambda i: (0, 0)),
        ],
        out_specs=pl.BlockSpec((_EDGE_BLOCK, m), lambda i: (i, 0)),
        out_shape=jax.ShapeDtypeStruct((n, m), jnp.int32),
    )(x, WA, bA.reshape(1, -1), WB, bB.reshape(1, -1))


def _linear_body(x_ref, w_ref, b_ref, o_ref):
    acc = (
        jnp.dot(x_ref[...], w_ref[...], preferred_element_type=jnp.float32)
        + b_ref[...]
    )
    o_ref[...] = acc.astype(o_ref.dtype)


def _tc_linear(x, W, b, block_rows, out_dtype=jnp.float32):
    n, k = x.shape
    m = W.shape[1]
    return pl.pallas_call(
        _linear_body,
        grid=(n // block_rows,),
        in_specs=[
            pl.BlockSpec((block_rows, k), lambda i: (i, 0)),
            pl.BlockSpec((k, m), lambda i: (0, 0)),
            pl.BlockSpec((1, m), lambda i: (0, 0)),
        ],
        out_specs=pl.BlockSpec((block_rows, m), lambda i: (i, 0)),
        out_shape=jax.ShapeDtypeStruct((n, m), out_dtype),
    )(x, W, b)


def _node_mlp_body(h_ref, agg_ref, w1_ref, b1_ref, w2_ref, b2_ref,
                   g_ref, be_ref, mu_ref, var_ref, eps_ref, o_ref):
    a = agg_ref[0] + agg_ref[1]
    z = (1.0 + eps_ref[0]) * h_ref[...] + a
    u = jnp.maximum(
        jnp.dot(z, w1_ref[...], preferred_element_type=jnp.float32) + b1_ref[...],
        0.0,
    )
    v = jnp.dot(u, w2_ref[...], preferred_element_type=jnp.float32) + b2_ref[...]
    scale = g_ref[...] * lax.rsqrt(var_ref[...] + BN_EPS_C)
    shift = be_ref[...] - mu_ref[...] * scale
    o_ref[...] = jnp.maximum(v * scale + shift, 0.0)


def _tc_node_mlp(h, agg2, W1i, b1i, W2i, b2i, gi, bei, mui, vari, epsi):
    n = h.shape[0]
    nb = n // _NODE_BLOCK
    row = lambda a: a.reshape(1, -1)
    return pl.pallas_call(
        _node_mlp_body,
        grid=(nb,),
        in_specs=[
            pl.BlockSpec((_NODE_BLOCK, HID), lambda i: (i, 0)),
            pl.BlockSpec((2, _NODE_BLOCK, HID), lambda i: (0, i, 0)),
            pl.BlockSpec((HID, 2 * HID), lambda i: (0, 0)),
            pl.BlockSpec((1, 2 * HID), lambda i: (0, 0)),
            pl.BlockSpec((2 * HID, HID), lambda i: (0, 0)),
            pl.BlockSpec((1, HID), lambda i: (0, 0)),
            pl.BlockSpec((1, HID), lambda i: (0, 0)),
            pl.BlockSpec((1, HID), lambda i: (0, 0)),
            pl.BlockSpec((1, HID), lambda i: (0, 0)),
            pl.BlockSpec((1, HID), lambda i: (0, 0)),
            pl.BlockSpec(memory_space=pltpu.SMEM),
        ],
        out_specs=pl.BlockSpec((_NODE_BLOCK, HID), lambda i: (i, 0)),
        out_shape=jax.ShapeDtypeStruct((n, HID), jnp.float32),
    )(h, agg2, W1i, row(b1i), W2i, row(b2i), row(gi), row(bei), row(mui),
      row(vari), epsi.reshape(1))


def _pool_body(h_ref, b3_ref, wf_ref, bf_ref, o_ref, sums_ref, cnts_ref):
    i = pl.program_id(0)
    nb = pl.num_programs(0)

    @pl.when(i == 0)
    def _():
        sums_ref[...] = jnp.zeros_like(sums_ref)
        cnts_ref[...] = jnp.zeros_like(cnts_ref)

    seg = b3_ref[0]
    gids = lax.broadcasted_iota(jnp.int32, (NGRAPH, seg.shape[1]), 0)
    onehot = (gids == seg).astype(jnp.float32)
    sums_ref[...] += jnp.dot(onehot, h_ref[...],
                             preferred_element_type=jnp.float32)
    cnts_ref[...] += jnp.sum(onehot, axis=1, keepdims=True)

    @pl.when(i == nb - 1)
    def _():
        pooled = sums_ref[...] / jnp.clip(cnts_ref[...], 1.0, None)
        o_ref[...] = (
            jnp.dot(pooled, wf_ref[...], preferred_element_type=jnp.float32)
            + bf_ref[...]
        )


def _tc_pool(h, batch3, W_final, b_final):
    n = h.shape[0]
    nb = n // _NODE_BLOCK
    return pl.pallas_call(
        _pool_body,
        grid=(nb,),
        in_specs=[
            pl.BlockSpec((_NODE_BLOCK, HID), lambda i: (i, 0)),
            pl.BlockSpec((1, 1, _NODE_BLOCK), lambda i: (i, 0, 0)),
            pl.BlockSpec((HID, HID), lambda i: (0, 0)),
            pl.BlockSpec((1, HID), lambda i: (0, 0)),
        ],
        out_specs=pl.BlockSpec((NGRAPH, HID), lambda i: (0, 0)),
        out_shape=jax.ShapeDtypeStruct((NGRAPH, HID), jnp.float32),
        scratch_shapes=[
            pltpu.VMEM((NGRAPH, HID), jnp.float32),
            pltpu.VMEM((NGRAPH, 1), jnp.float32),
        ],
    )(h, batch3, W_final, b_final.reshape(1, -1))


_SC_MESH = plsc.VectorSubcoreMesh(core_axis_name="c", subcore_axis_name="s")

_SC_CP = pltpu.CompilerParams()
if "needs_layout_passes" in pltpu.CompilerParams.__dataclass_fields__:
    _SC_CP = dataclasses.replace(_SC_CP, needs_layout_passes=False)


_NPW = -(-_NCHUNK // _NWORKER)
_NPW3 = _NPW + (-_NPW) % 3


@functools.partial(
    pl.kernel,
    out_type=jax.ShapeDtypeStruct((2, N_NODES, HID), jnp.float32),
    mesh=_SC_MESH,
    compiler_params=_SC_CP,
    scratch_types=[
        pltpu.VMEM((_CHUNK,), jnp.int32),
        pltpu.VMEM((_CHUNK,), jnp.int32),
        pltpu.VMEM((_CHUNK,), jnp.int32),
        pltpu.VMEM((_CHUNK,), jnp.int32),
        pltpu.VMEM((_CHUNK,), jnp.int32),
        pltpu.VMEM((_CHUNK,), jnp.int32),
        pltpu.VMEM((_CHUNK,), jnp.int32),
        pltpu.VMEM((_CHUNK,), jnp.int32),
        pltpu.VMEM((_CHUNK,), jnp.int32),
        pltpu.VMEM((3, _CHUNK, HID), jnp.float32),
        pltpu.VMEM((3, _CHUNK, HID // 2), jnp.int32),
        pltpu.VMEM_SHARED((N_NODES, HID), jnp.float32),
        pltpu.SemaphoreType.DMA,
        pltpu.SemaphoreType.DMA,
        pltpu.SemaphoreType.DMA,
        pltpu.SemaphoreType.DMA,
        pltpu.SemaphoreType.DMA,
        pltpu.SemaphoreType.DMA,
        pltpu.SemaphoreType.DMA,
        pltpu.SemaphoreType.DMA,
        pltpu.SemaphoreType.DMA,
        pltpu.SemaphoreType.DMA,
        pltpu.SemaphoreType.DMA,
        pltpu.SemaphoreType.DMA,
    ],
)
def _sc_msg_agg(h_hbm, e_hbm, src_hbm, dst_hbm, zeros_hbm, out_hbm,
                sr0, sr1, sr2, dr0, dr1, dr2, dn0, dn1, dn2,
                rows_v, e_v, agg_sh,
                si0, si1, si2, sg0, sg1, sg2, se0, se1, se2, ss0, ss1, ss2):
    srcin = (sr0, sr1, sr2)
    dstin = (dr0, dr1, dr2)
    dsts = (dn0, dn1, dn2)
    cid = lax.axis_index("c")
    sid = lax.axis_index("s")
    wid = cid * _NTILE + sid
    row0 = sid * _ROWS_MAIN

    si = (si0, si1, si2)
    sg = (sg0, sg1, sg2)
    se = (se0, se1, se2)
    ss = (ss0, ss1, ss2)

    pltpu.sync_copy(zeros_hbm, agg_sh.at[pl.ds(row0, _ROWS_MAIN)])

    @pl.when(sid == _NTILE - 1)
    def _():
        pltpu.sync_copy(zeros_hbm.at[pl.ds(0, _TAIL)],
                        agg_sh.at[pl.ds(_TAIL0, _TAIL)])

    plsc.subcore_barrier()

    def chunk_of(q):
        return q * _NWORKER + wid

    def idx_copies(q, b):
        base = chunk_of(q) * _CHUNK
        return (
            pltpu.make_async_copy(src_hbm.at[pl.ds(base, _CHUNK)],
                                  srcin[b], si[b]),
            pltpu.make_async_copy(dst_hbm.at[pl.ds(base, _CHUNK)],
                                  dstin[b], si[b]),
        )

    def issue_idx(q, b):
        for cp in idx_copies(q, b):
            cp.start()

    def wait_idx(q, b):
        for cp in idx_copies(q, b):
            cp.wait()

    def issue_body(q, b):
        pltpu.async_copy(h_hbm.at[srcin[b]], rows_v.at[b], sg[b])
        pltpu.async_copy(
            e_hbm.at[pl.ds(chunk_of(q) * _CHUNK, _CHUNK)], e_v.at[b], se[b])

    def wait_scatter(b):
        pltpu.make_async_copy(
            rows_v.at[b], agg_sh.at[dsts[b]], ss[b]).wait()

    issue_idx(0, 0)
    wait_idx(0, 0)
    issue_body(0, 0)
    issue_idx(1, 1)
    issue_idx(2, 2)

    def substep(q, b):
        b1 = (b + 1) % 3

        @pl.when(chunk_of(q + 1) < _NCHUNK)
        def _():
            wait_idx(q + 1, b1)

            @pl.when(q >= 2)
            def _():
                wait_scatter(b1)

            issue_body(q + 1, b1)

        @pl.when(chunk_of(q) < _NCHUNK)
        def _():
            pltpu.make_async_copy(h_hbm.at[srcin[b]], rows_v.at[b],
                                  sg[b]).wait()
            pltpu.make_async_copy(
                e_hbm.at[pl.ds(chunk_of(q) * _CHUNK, _CHUNK)], e_v.at[b],
                se[b]).wait()

            for k in range(_CHUNK // 16):
                sl = pl.ds(k * 16, 16)
                dsts[b][sl] = dstin[b][sl]

            @pl.loop(0, _CHUNK)
            def _(r):
                for k in range(HID // 32):
                    ew = e_v[b, r, pl.ds(k * 16, 16)]
                    ebf = plsc.bitcast(ew, jnp.bfloat16)
                    e0, e1 = plsc.unpack(
                        ebf, format=plsc.PackFormat.INTERLEAVED)
                    lo = pl.ds(k * 32, 16)
                    hi = pl.ds(k * 32 + 16, 16)
                    rows_v[b, r, lo] = jnp.maximum(rows_v[b, r, lo] + e0, 0.0)
                    rows_v[b, r, hi] = jnp.maximum(rows_v[b, r, hi] + e1, 0.0)

            pltpu.async_copy(rows_v.at[b], agg_sh.at[dsts[b]],
                             ss[b], add=True)

        @pl.when(chunk_of(q + 3) < _NCHUNK)
        def _():
            issue_idx(q + 3, b)

    @pl.loop(0, _NPW3, step=3)
    def _(q):
        substep(q, 0)
        substep(q + 1, 1)
        substep(q + 2, 2)

    for b in range(3):
        wait_scatter(b)

    plsc.subcore_barrier()
    pltpu.sync_copy(agg_sh.at[pl.ds(row0, _ROWS_MAIN)],
                    out_hbm.at[cid, pl.ds(row0, _ROWS_MAIN)])

    @pl.when(sid == _NTILE - 1)
    def _():
        pltpu.sync_copy(agg_sh.at[pl.ds(_TAIL0, _TAIL)],
                        out_hbm.at[cid, pl.ds(_TAIL0, _TAIL)])


def kernel(x, edge_index, batch, edge_attr, W_init, b_init, eps,
           W_edge, b_edge, W1, b1, W2, b2, bn_gamma, bn_beta, bn_mean,
           bn_var, W_final, b_final):
    eidx = edge_index.astype(jnp.int32)
    src_ids = eidx[0]
    dst_ids = eidx[1]
    batch3 = batch.astype(jnp.int32).reshape(
        N_NODES // _NODE_BLOCK, 1, _NODE_BLOCK)
    zeros = jnp.zeros((_ROWS_MAIN, HID), jnp.float32)

    cols_a = []
    cols_b = []
    for g in range(HID // 32):
        cols_a.extend(range(g * 32, g * 32 + 16))
        cols_b.extend(range(g * 32 + 16, g * 32 + 32))
    cols_a = jnp.asarray(cols_a, dtype=jnp.int32)
    cols_b = jnp.asarray(cols_b, dtype=jnp.int32)

    h = _tc_linear(x, W_init, b_init.reshape(1, -1), _NODE_BLOCK)
    gate = x[0, 0]
    for i in range(W_edge.shape[0]):
        ea, _ = lax.optimization_barrier((edge_attr, gate))
        e = _tc_edge_pack(ea, W_edge[i][:, cols_a], b_edge[i][cols_a],
                          W_edge[i][:, cols_b], b_edge[i][cols_b])
        agg2 = _sc_msg_agg(h, e, src_ids, dst_ids, zeros)
        gate = agg2[0, 0, 0]
        h = _tc_node_mlp(h, agg2, W1[i], b1[i], W2[i], b2[i],
                         bn_gamma[i], bn_beta[i], bn_mean[i], bn_var[i],
                         eps[i])
    return _tc_pool(h, batch3, W_final, b_final)
